# Initial kernel scaffold; baseline (speedup 1.0000x reference)
import jax
import jax.numpy as jnp
from jax.experimental import pallas as pl


def kernel(x, mask_token):
    raise NotImplementedError("write your pallas kernel here")


# speedup vs baseline: 1.5623x; 1.5623x over previous
import functools

import jax
import jax.numpy as jnp
import numpy as np
from jax import lax
from jax.experimental import pallas as pl
from jax.experimental.pallas import tpu as pltpu

_PATCH_SIZE = 16
_MASK_RATIO = 0.75


@functools.lru_cache(maxsize=None)
def _patch_mask_np(B, nH, nW, nD):
    with jax.ensure_compile_time_eval():
        return _patch_mask_eval(B, nH, nW, nD)


def _patch_mask_eval(B, nH, nW, nD):
    n_patches = nH * nW * nD
    n_masked = int(n_patches * _MASK_RATIO)
    key = jax.random.key(42)
    keys = jax.random.split(key, B)
    rows = []
    for b in range(B):
        perm = jax.random.permutation(keys[b], n_patches)
        idx = perm[:n_masked]
        flat = jnp.zeros((n_patches,), dtype=bool).at[idx].set(True)
        rows.append(flat.reshape(nH, nW, nD))
    pm = jnp.stack(rows, axis=0)
    return np.asarray(jax.device_get(pm)).astype(np.float32)


def _body(pm_ref, tok_ref, x_ref, out_ref, vm_ref):
    pm = pm_ref[0, 0]
    tok = tok_ref[0, 0]
    e = (lax.broadcasted_iota(jnp.int32, (8, 128), 1) // _PATCH_SIZE
         == lax.broadcasted_iota(jnp.int32, (8, 128), 0)).astype(jnp.float32)
    et = (lax.broadcasted_iota(jnp.int32, (128, 8), 0) // _PATCH_SIZE
          == lax.broadcasted_iota(jnp.int32, (128, 8), 1)).astype(jnp.float32)
    a = jnp.dot(pm, e, preferred_element_type=jnp.float32)
    vm2 = jnp.dot(et, a, preferred_element_type=jnp.float32)
    vmb = vm2 > 0.5
    xblk = x_ref[...]
    sel = jnp.broadcast_to(vmb[None, None, None], xblk.shape)
    out_ref[...] = jnp.where(sel, tok, xblk)
    vm_ref[...] = jnp.broadcast_to(vmb[None, None], vm_ref.shape)


def kernel(x, mask_token):
    B, C, H, W, D = x.shape
    ps = _PATCH_SIZE
    nH, nW, nD = H // ps, W // ps, D // ps
    pm = jnp.asarray(_patch_mask_np(B, nH, nW, nD))
    tok = mask_token.reshape(1, 1)

    grid = (B, nH)
    masked_x, voxel_mask = pl.pallas_call(
        _body,
        grid=grid,
        in_specs=[
            pl.BlockSpec((1, 1, nW, nD), lambda b, h: (b, h, 0, 0)),
            pl.BlockSpec(memory_space=pltpu.SMEM),
            pl.BlockSpec((1, C, ps, W, D), lambda b, h: (b, 0, h, 0, 0)),
        ],
        out_specs=[
            pl.BlockSpec((1, C, ps, W, D), lambda b, h: (b, 0, h, 0, 0)),
            pl.BlockSpec((1, ps, W, D), lambda b, h: (b, h, 0, 0)),
        ],
        out_shape=[
            jax.ShapeDtypeStruct((B, C, H, W, D), x.dtype),
            jax.ShapeDtypeStruct((B, H, W, D), jnp.bool_),
        ],
    )(pm, tok, x)
    return masked_x, voxel_mask
